# Initial kernel scaffold; baseline (speedup 1.0000x reference)
import jax
import jax.numpy as jnp
from jax.experimental import pallas as pl


def kernel(node_attr, edge_index, edge_feature, edge_vector, W1, b1, W2, b2, W, b, Wo, bo, Wo1, bo1):
    raise NotImplementedError("write your pallas kernel here")


# speedup vs baseline: 23.8560x; 23.8560x over previous
import jax
import jax.numpy as jnp
from jax import lax
from jax.experimental import pallas as pl
from jax.experimental.pallas import tpu as pltpu
from jax.experimental.pallas import tpu_sc as plsc

N = 10000
E = 160000
NH = 4
HD = 16
F = NH * HD
NC, NS = 2, 16
NW = NC * NS
CH = 128
KCH = 40
EW = CH * KCH
EP = EW * NW
GB = 4
GROUP = CH * GB
NP = 10240
NROWS = NP // NS

BQ = 2000
BE = 4096
BS = 2048
BN = 2048


def _elu(x):
    return jnp.where(x > 0, x, jnp.exp(x) - 1.0)


def _qkv_body(x_ref, w1_ref, b1_ref, w2_ref, b2_ref, w_ref, b_ref, o_ref):
    h1 = _elu(x_ref[...] * w1_ref[...] + b1_ref[...])
    h2 = jnp.dot(h1, w2_ref[...], preferred_element_type=jnp.float32) + b2_ref[...]
    o_ref[...] = jnp.dot(h2, w_ref[...], preferred_element_type=jnp.float32) + b_ref[...]


def _emat_body(ef_ref, w2_ref, b2_ref, w_ref, b_ref, o_ref):
    ea = _elu(jnp.dot(ef_ref[...], w2_ref[...], preferred_element_type=jnp.float32)
              + b2_ref[...])
    o_ref[...] = jnp.dot(ea, w_ref[...], preferred_element_type=jnp.float32) + b_ref[...]


def _score_body(ks_ref, qd_ref, em_ref, smat_ref, cmat_ref, msg_ref, zc_ref):
    i = pl.program_id(0)
    ks = ks_ref[...]
    t = ks * qd_ref[...] * em_ref[...]
    s = jnp.dot(t, smat_ref[...], preferred_element_type=jnp.float32) * 0.25
    sc = jnp.exp(jnp.clip(s, -5.0, 5.0))
    rows = i * BS + lax.broadcasted_iota(jnp.int32, (BS, 1), 0)
    sc = jnp.where(rows < E, sc, 0.0)
    msg_ref[...] = ks * sc
    zc_ref[...] = jnp.dot(sc, cmat_ref[...], preferred_element_type=jnp.float32)


def _out_body(wv_ref, z_ref, dmat_ref, wo_ref, bo_ref, wo1_ref, bo1_ref,
              o_ref, c_ref):
    wv = wv_ref[0] + wv_ref[1]
    z = z_ref[0] + z_ref[1]
    zb = jnp.dot(z, dmat_ref[...], preferred_element_type=jnp.float32)
    h = wv / (zb + 1e-6)
    ho = _elu(jnp.dot(h, wo_ref[...], preferred_element_type=jnp.float32) + bo_ref[...])
    o_ref[...] = jnp.dot(ho, wo1_ref[...], preferred_element_type=jnp.float32) + bo1_ref[...]
    c_ref[...] = (jnp.dot(_elu(bo_ref[...]), wo1_ref[...],
                          preferred_element_type=jnp.float32) + bo1_ref[...])


def _gather_body(qkv_hbm, src_hbm, dst_hbm, ks_out, qd_out,
                 sidx, didx, kbuf, qbuf, sem):
    wid = lax.axis_index("s") * NC + lax.axis_index("c")
    pltpu.sync_copy(src_hbm.at[pl.ds(wid * KCH, KCH)], sidx)
    pltpu.sync_copy(dst_hbm.at[pl.ds(wid * KCH, KCH)], didx)

    def group(g, carry):
        cps = []
        for j in range(GB):
            ch = g * GB + j
            cps.append(pltpu.async_copy(
                qkv_hbm.at[sidx.at[ch]], kbuf.at[pl.ds(j * CH, CH)], sem))
            cps.append(pltpu.async_copy(
                qkv_hbm.at[didx.at[ch]], qbuf.at[pl.ds(j * CH, CH)], sem))
        for cp in cps:
            cp.wait()
        base = wid * EW + g * GROUP
        pltpu.sync_copy(kbuf, ks_out.at[pl.ds(base, GROUP)])
        pltpu.sync_copy(qbuf, qd_out.at[pl.ds(base, GROUP)])
        return carry

    lax.fori_loop(0, KCH // GB, group, 0)


def _scatter_body(msg_hbm, zc_hbm, dst_hbm, z64_hbm, z16_hbm,
                  wv_out, zz_out,
                  didx, mbuf, zbuf, zvm64, zvm16, wv_sh, z_sh):
    cid = lax.axis_index("c")
    sid = lax.axis_index("s")
    wid = sid * NC + cid
    pltpu.sync_copy(z64_hbm, zvm64)
    pltpu.sync_copy(z16_hbm, zvm16)
    pltpu.sync_copy(zvm64, wv_sh.at[pl.ds(sid * NROWS, NROWS)])
    pltpu.sync_copy(zvm16, z_sh.at[pl.ds(sid * NROWS, NROWS)])
    pltpu.sync_copy(dst_hbm.at[pl.ds(wid * KCH, KCH)], didx)
    plsc.subcore_barrier()

    def chunk(ch, carry):
        base = wid * EW + ch * CH
        pltpu.sync_copy(msg_hbm.at[pl.ds(base, CH)], mbuf)
        pltpu.sync_copy(zc_hbm.at[pl.ds(base, CH)], zbuf)
        pltpu.sync_copy(mbuf, wv_sh.at[didx.at[ch]], add=True)
        pltpu.sync_copy(zbuf, z_sh.at[didx.at[ch]], add=True)
        return carry

    lax.fori_loop(0, KCH, chunk, 0)
    plsc.subcore_barrier()
    out_base = cid * NP + sid * NROWS
    pltpu.sync_copy(wv_sh.at[pl.ds(sid * NROWS, NROWS)],
                    wv_out.at[pl.ds(out_base, NROWS)])
    pltpu.sync_copy(z_sh.at[pl.ds(sid * NROWS, NROWS)],
                    zz_out.at[pl.ds(out_base, NROWS)])


def kernel(node_attr, edge_index, edge_feature, edge_vector,
           W1, b1, W2, b2, W, b, Wo, bo, Wo1, bo1):
    f32 = jnp.float32
    b1r = b1.reshape(1, -1)
    b2r = b2.reshape(1, -1)
    br = b.reshape(1, -1)
    bor = bo.reshape(1, -1)
    bo1r = bo1.reshape(1, -1)

    x10k = edge_vector.reshape(-1)[:N].reshape(N, 1)
    pad = jnp.zeros((EP - E,), jnp.int32)
    src2d = jnp.concatenate([edge_index[0], pad]).reshape(EP // CH, CH)
    dst2d = jnp.concatenate([edge_index[1], pad]).reshape(EP // CH, CH)
    ef_pad = jnp.zeros((EP, 4), f32).at[:E].set(edge_feature)
    dgrp = jnp.arange(F) // HD
    smat = (dgrp[:, None] == dgrp[None, :]).astype(f32)
    cmat = (jnp.arange(F)[:, None] == (jnp.arange(HD) * HD)[None, :]).astype(f32)
    dmat = (jnp.arange(HD)[:, None] == dgrp[None, :]).astype(f32)
    z64 = jnp.zeros((NROWS, F), f32)
    z16 = jnp.zeros((NROWS, HD), f32)

    full = lambda shape: pl.BlockSpec(shape, lambda i: (0, 0))
    qkv = pl.pallas_call(
        _qkv_body,
        grid=(N // BQ,),
        in_specs=[pl.BlockSpec((BQ, 1), lambda i: (i, 0)),
                  full((1, 4)), full((1, 4)), full((4, HD)), full((1, HD)),
                  full((HD, F)), full((1, F))],
        out_specs=pl.BlockSpec((BQ, F), lambda i: (i, 0)),
        out_shape=jax.ShapeDtypeStruct((N, F), f32),
    )(x10k, W1, b1r, W2, b2r, W, br)

    emat = pl.pallas_call(
        _emat_body,
        grid=(EP // BE,),
        in_specs=[pl.BlockSpec((BE, 4), lambda i: (i, 0)),
                  full((4, HD)), full((1, HD)), full((HD, F)), full((1, F))],
        out_specs=pl.BlockSpec((BE, F), lambda i: (i, 0)),
        out_shape=jax.ShapeDtypeStruct((EP, F), f32),
    )(ef_pad, W2, b2r, W, br)

    mesh = plsc.VectorSubcoreMesh(core_axis_name="c", subcore_axis_name="s",
                                  num_cores=NC, num_subcores=NS)
    ks, qd = pl.kernel(
        _gather_body,
        out_type=[jax.ShapeDtypeStruct((EP, F), f32),
                  jax.ShapeDtypeStruct((EP, F), f32)],
        mesh=mesh,
        compiler_params=pltpu.CompilerParams(use_tc_tiling_on_sc=False),
        scratch_types=[pltpu.VMEM((KCH, CH), jnp.int32),
                       pltpu.VMEM((KCH, CH), jnp.int32),
                       pltpu.VMEM((GROUP, F), f32),
                       pltpu.VMEM((GROUP, F), f32),
                       pltpu.SemaphoreType.DMA],
    )(qkv, src2d, dst2d)

    msg, zc = pl.pallas_call(
        _score_body,
        grid=(EP // BS,),
        in_specs=[pl.BlockSpec((BS, F), lambda i: (i, 0)),
                  pl.BlockSpec((BS, F), lambda i: (i, 0)),
                  pl.BlockSpec((BS, F), lambda i: (i, 0)),
                  full((F, F)), full((F, HD))],
        out_specs=[pl.BlockSpec((BS, F), lambda i: (i, 0)),
                   pl.BlockSpec((BS, HD), lambda i: (i, 0))],
        out_shape=[jax.ShapeDtypeStruct((EP, F), f32),
                   jax.ShapeDtypeStruct((EP, HD), f32)],
    )(ks, qd, emat, smat, cmat)

    wv2, zz2 = pl.kernel(
        _scatter_body,
        out_type=[jax.ShapeDtypeStruct((NC * NP, F), f32),
                  jax.ShapeDtypeStruct((NC * NP, HD), f32)],
        mesh=mesh,
        compiler_params=pltpu.CompilerParams(use_tc_tiling_on_sc=False),
        scratch_types=[pltpu.VMEM((KCH, CH), jnp.int32),
                       pltpu.VMEM((CH, F), f32),
                       pltpu.VMEM((CH, HD), f32),
                       pltpu.VMEM((NROWS, F), f32),
                       pltpu.VMEM((NROWS, HD), f32),
                       pltpu.VMEM_SHARED((NP, F), f32),
                       pltpu.VMEM_SHARED((NP, HD), f32)],
    )(msg, zc, dst2d, z64, z16)

    h10k, cval = pl.pallas_call(
        _out_body,
        grid=(NP // BN,),
        in_specs=[pl.BlockSpec((NC, BN, F), lambda i: (0, i, 0)),
                  pl.BlockSpec((NC, BN, HD), lambda i: (0, i, 0)),
                  full((HD, F)), full((F, HD)), full((1, HD)),
                  full((HD, 1)), full((1, 1))],
        out_specs=[pl.BlockSpec((BN, 1), lambda i: (i, 0)),
                   pl.BlockSpec((1, 1), lambda i: (0, 0))],
        out_shape=[jax.ShapeDtypeStruct((NP, 1), f32),
                   jax.ShapeDtypeStruct((1, 1), f32)],
    )(wv2.reshape(NC, NP, F), zz2.reshape(NC, NP, HD),
      dmat, Wo, bor, Wo1, bo1r)

    flat = jnp.concatenate([h10k.reshape(-1)[:N],
                            jnp.broadcast_to(cval.reshape(()), (3 * E - N,))])
    return flat.reshape(E, 3)
